# baseline (speedup 1.0000x reference)
import jax
import jax.numpy as jnp
from jax.experimental import pallas as pl

B = 128
N = 512
L = 10
LP = 16
TOPK = 20
E = 64
BLP = B * LP
BB = 8
OUTW = L + 2 * E


def _pre_body(x_ref, out_ref):
    out_ref[...] = jnp.zeros((N, BB * LP), jnp.float32)
    for bb in range(BB):
        out_ref[:, bb * LP : bb * LP + L] = x_ref[bb]


def _main_body(xp_ref, s_ref, t_ref, W1_ref, W2_ref, W3_ref, b_ref, agg_ref):
    X = xp_ref[...]
    lane = jax.lax.broadcasted_iota(jnp.int32, (1, BLP), 1) % LP

    grow = jax.lax.broadcasted_iota(jnp.int32, (3 * N, N), 0)
    gcol = jax.lax.broadcasted_iota(jnp.int32, (3 * N, N), 1)
    G = [jnp.where(grow == 3 * gcol + k, 1.0, 0.0) for k in range(3)]

    def shift(V, s):
        sh = jnp.concatenate([jnp.zeros((N, s), V.dtype), V[:, : BLP - s]], axis=1)
        return jnp.where(lane >= s, sh, 0.0)

    def layer(V, W_ref, bias, d):
        W2d = W_ref[...]
        Wk = [jnp.dot(W2d, G[k], preferred_element_type=jnp.float32) for k in range(3)]
        acc = jnp.dot(Wk[2], V, preferred_element_type=jnp.float32)
        acc += jnp.dot(Wk[1], shift(V, d), preferred_element_type=jnp.float32)
        acc += jnp.dot(Wk[0], shift(V, 2 * d), preferred_element_type=jnp.float32)
        return jax.nn.relu(acc + bias)

    b = b_ref[...]
    Z = layer(X, W1_ref, b[0][:, None], 1)
    Z = layer(Z, W2_ref, b[1][:, None], 2)
    Z = layer(Z, W3_ref, b[2][:, None], 4)
    Z = jax.nn.relu(Z + X)

    s = s_ref[...]
    t = t_ref[...]
    ns = s * jax.lax.rsqrt(jnp.sum(s * s, axis=1, keepdims=True))
    nt = t * jax.lax.rsqrt(jnp.sum(t * t, axis=1, keepdims=True))
    c = jax.nn.relu(jnp.dot(nt, ns.T, preferred_element_type=jnp.float32))
    col = jax.lax.broadcasted_iota(jnp.int32, (N, N), 1)
    row = jax.lax.broadcasted_iota(jnp.int32, (N, N), 0)
    S = jnp.where(col == row, -jnp.inf, c)

    A = jnp.zeros((N, N), jnp.float32)
    for _ in range(TOPK):
        v = jnp.max(S, axis=1, keepdims=True)
        m = S == v
        idx = jnp.where(m, col, N)
        jmin = jnp.min(idx, axis=1, keepdims=True)
        first = col == jmin
        A = jnp.where(first, 1.0, A)
        S = jnp.where(first, -jnp.inf, S)

    agg_ref[...] = jax.nn.relu(jnp.dot(A, Z, preferred_element_type=jnp.float32))


def _asm_body(agg_ref, s_ref, t_ref, out_ref):
    se = s_ref[...]
    te = t_ref[...]
    for bb in range(BB):
        out_ref[bb, :, 0:L] = agg_ref[:, bb * LP : bb * LP + L]
        out_ref[bb, :, L : L + E] = se
        out_ref[bb, :, L + E : OUTW] = te


def kernel(x, s_emb, t_emb, W1, b1, W2, b2, W3, b3):
    Ws = [W.reshape(N, 3 * N) for W in (W1, W2, W3)]
    bs = jnp.stack([b1, b2, b3], axis=0)

    xp = pl.pallas_call(
        _pre_body,
        grid=(B // BB,),
        in_specs=[pl.BlockSpec((BB, N, L), lambda i: (i, 0, 0))],
        out_specs=pl.BlockSpec((N, BB * LP), lambda i: (0, i)),
        out_shape=jax.ShapeDtypeStruct((N, BLP), jnp.float32),
    )(x)

    agg = pl.pallas_call(
        _main_body,
        out_shape=jax.ShapeDtypeStruct((N, BLP), jnp.float32),
    )(xp, s_emb, t_emb, Ws[0], Ws[1], Ws[2], bs)

    out3 = pl.pallas_call(
        _asm_body,
        grid=(B // BB,),
        in_specs=[
            pl.BlockSpec((N, BB * LP), lambda i: (0, i)),
            pl.BlockSpec((N, E), lambda i: (0, 0)),
            pl.BlockSpec((N, E), lambda i: (0, 0)),
        ],
        out_specs=pl.BlockSpec((BB, N, OUTW), lambda i: (i, 0, 0)),
        out_shape=jax.ShapeDtypeStruct((B, N, OUTW), jnp.float32),
    )(agg, s_emb, t_emb)

    return out3.reshape(B * N, OUTW)


# speedup vs baseline: 1.6678x; 1.6678x over previous
import jax
import jax.numpy as jnp
from jax.experimental import pallas as pl

B = 128
N = 512
L = 10
TOPK = 20
E = 64
BB = 8
OUTW = L + 2 * E

_NT = (((1,), (1,)), ((), ()))
_TN = (((0,), (0,)), ((), ()))


def _main_body(x_ref, s_ref, t_ref, W1_ref, W2_ref, W3_ref, b_ref, agg_ref):
    bias = b_ref[...]

    def layer(V, W_ref, j, d):
        acc = [None] * L
        for k in range(3):
            Wk = W_ref[k]
            s = (2 - k) * d
            for t in range(L):
                if t - s < 0:
                    continue
                p = jax.lax.dot_general(V[t - s], Wk, _NT,
                                        preferred_element_type=jnp.float32)
                acc[t] = p if acc[t] is None else acc[t] + p
        return [jax.nn.relu(a + bias[j][None, :]) for a in acc]

    X = [x_ref[t] for t in range(L)]
    Z = layer(X, W1_ref, 0, 1)
    Z = layer(Z, W2_ref, 1, 2)
    Z = layer(Z, W3_ref, 2, 4)
    Z = [jax.nn.relu(z + x) for z, x in zip(Z, X)]

    sT = s_ref[...]
    tT = t_ref[...]
    nsT = sT * jax.lax.rsqrt(jnp.sum(sT * sT, axis=0, keepdims=True))
    ntT = tT * jax.lax.rsqrt(jnp.sum(tT * tT, axis=0, keepdims=True))
    c = jax.nn.relu(jax.lax.dot_general(ntT, nsT, _TN,
                                        preferred_element_type=jnp.float32))
    col = jax.lax.broadcasted_iota(jnp.int32, (N, N), 1)
    row = jax.lax.broadcasted_iota(jnp.int32, (N, N), 0)
    S = jnp.where(col == row, -jnp.inf, c)

    A = jnp.zeros((N, N), jnp.float32)
    for _ in range(TOPK):
        v = jnp.max(S, axis=1, keepdims=True)
        m = S == v
        idx = jnp.where(m, col, N)
        jmin = jnp.min(idx, axis=1, keepdims=True)
        first = col == jmin
        A = jnp.where(first, 1.0, A)
        S = jnp.where(first, -jnp.inf, S)

    for t in range(L):
        agg_ref[t] = jax.nn.relu(
            jax.lax.dot_general(Z[t], A, _NT, preferred_element_type=jnp.float32))


def _asm_body(agg_ref, s_ref, t_ref, out_ref):
    out_ref[0:L] = agg_ref[...]
    out_ref[L : L + E] = jax.lax.broadcast_in_dim(s_ref[...], (E, BB, N), (0, 2))
    out_ref[L + E : OUTW] = jax.lax.broadcast_in_dim(t_ref[...], (E, BB, N), (0, 2))


def kernel(x, s_emb, t_emb, W1, b1, W2, b2, W3, b3):
    xr = x.transpose(2, 0, 1)
    Ws = [W.transpose(2, 0, 1) for W in (W1, W2, W3)]
    sT = s_emb.T
    tT = t_emb.T
    bs = jnp.stack([b1, b2, b3], axis=0)

    agg = pl.pallas_call(
        _main_body,
        out_shape=jax.ShapeDtypeStruct((L, B, N), jnp.float32),
    )(xr, sT, tT, Ws[0], Ws[1], Ws[2], bs)

    out3 = pl.pallas_call(
        _asm_body,
        grid=(B // BB,),
        in_specs=[
            pl.BlockSpec((L, BB, N), lambda i: (0, i, 0)),
            pl.BlockSpec((E, N), lambda i: (0, 0)),
            pl.BlockSpec((E, N), lambda i: (0, 0)),
        ],
        out_specs=pl.BlockSpec((OUTW, BB, N), lambda i: (0, i, 0)),
        out_shape=jax.ShapeDtypeStruct((OUTW, B, N), jnp.float32),
    )(agg, sT, tT)

    return out3.reshape(OUTW, B * N).T


# speedup vs baseline: 4.7435x; 2.8441x over previous
import jax
import jax.numpy as jnp
from jax.experimental import pallas as pl

B = 128
N = 512
L = 10
TOPK = 20
E = 64
BB = 8
OUTW = L + 2 * E

_NT = (((1,), (1,)), ((), ()))
_TN = (((0,), (0,)), ((), ()))


def _main_body(x_ref, s_ref, t_ref, W1_ref, W2_ref, W3_ref, b_ref, agg_ref):
    bias = b_ref[...]
    X = x_ref[...]

    def shift(V, s):
        r = s * B
        return jnp.concatenate([jnp.zeros((r, N), V.dtype), V[: L * B - r]], axis=0)

    def layer(V, W_ref, j, d):
        acc = jax.lax.dot_general(V, W_ref[2], _NT, preferred_element_type=jnp.float32)
        acc += jax.lax.dot_general(shift(V, d), W_ref[1], _NT,
                                   preferred_element_type=jnp.float32)
        acc += jax.lax.dot_general(shift(V, 2 * d), W_ref[0], _NT,
                                   preferred_element_type=jnp.float32)
        return jax.nn.relu(acc + bias[j][None, :])

    Z = layer(X, W1_ref, 0, 1)
    Z = layer(Z, W2_ref, 1, 2)
    Z = layer(Z, W3_ref, 2, 4)
    Z = jax.nn.relu(Z + X)

    sT = s_ref[...]
    tT = t_ref[...]
    nsT = sT * jax.lax.rsqrt(jnp.sum(sT * sT, axis=0, keepdims=True))
    ntT = tT * jax.lax.rsqrt(jnp.sum(tT * tT, axis=0, keepdims=True))
    c = jax.nn.relu(jax.lax.dot_general(ntT, nsT, _TN,
                                        preferred_element_type=jnp.float32))
    col = jax.lax.broadcasted_iota(jnp.int32, (N, N), 1)
    row = jax.lax.broadcasted_iota(jnp.int32, (N, N), 0)
    S = jnp.where(col == row, -jnp.inf, c)

    A = jnp.zeros((N, N), jnp.float32)
    for _ in range(TOPK):
        v = jnp.max(S, axis=1, keepdims=True)
        m = S == v
        idx = jnp.where(m, col, N)
        jmin = jnp.min(idx, axis=1, keepdims=True)
        first = col == jmin
        A = jnp.where(first, 1.0, A)
        S = jnp.where(first, -jnp.inf, S)

    agg_ref[...] = jax.nn.relu(
        jax.lax.dot_general(Z, A, _NT, preferred_element_type=jnp.float32))


def _asm_body(agg_ref, s_ref, t_ref, out_ref):
    se = s_ref[...]
    te = t_ref[...]
    a = agg_ref[...]
    for bb in range(BB):
        out_ref[0:L, bb * N : (bb + 1) * N] = a[:, bb, :]
        out_ref[L : L + E, bb * N : (bb + 1) * N] = se
        out_ref[L + E : OUTW, bb * N : (bb + 1) * N] = te


def kernel(x, s_emb, t_emb, W1, b1, W2, b2, W3, b3):
    xf = x.transpose(2, 0, 1).reshape(L * B, N)
    Ws = [W.transpose(2, 0, 1) for W in (W1, W2, W3)]
    sT = s_emb.T
    tT = t_emb.T
    bs = jnp.stack([b1, b2, b3], axis=0)

    agg = pl.pallas_call(
        _main_body,
        out_shape=jax.ShapeDtypeStruct((L * B, N), jnp.float32),
    )(xf, sT, tT, Ws[0], Ws[1], Ws[2], bs)

    agg3 = agg.reshape(L, B, N)

    out2 = pl.pallas_call(
        _asm_body,
        grid=(B // BB,),
        in_specs=[
            pl.BlockSpec((L, BB, N), lambda i: (0, i, 0)),
            pl.BlockSpec((E, N), lambda i: (0, 0)),
            pl.BlockSpec((E, N), lambda i: (0, 0)),
        ],
        out_specs=pl.BlockSpec((OUTW, BB * N), lambda i: (0, i)),
        out_shape=jax.ShapeDtypeStruct((OUTW, B * N), jnp.float32),
    )(agg3, sT, tT)

    return out2.T
